# baseline (speedup 1.0000x reference)
import functools

import jax
import jax.numpy as jnp
from jax import lax
from jax.experimental import pallas as pl
from jax.experimental.pallas import tpu as pltpu
from jax.experimental.pallas import tpu_sc as plsc

BATCH = 16384
N_FIELDS = 26
DIM = 16
LANES = 16

NUM_CORES = 2
NUM_SUBCORES = 16
NW = NUM_CORES * NUM_SUBCORES

ROWS_PER_DMA = 128
DMAS_PER_CHUNK = 13
CHUNK = ROWS_PER_DMA * DMAS_PER_CHUNK
NCHUNKS = (BATCH * N_FIELDS) // (NW * CHUNK)


def _sc_gather(ids, table, off_tiled):
    mesh = plsc.VectorSubcoreMesh(core_axis_name="c", subcore_axis_name="s")

    @functools.partial(
        pl.kernel,
        mesh=mesh,
        out_type=jax.ShapeDtypeStruct(
            (NW, NCHUNKS, DMAS_PER_CHUNK, ROWS_PER_DMA, DIM), jnp.float32
        ),
        scratch_types=[
            pltpu.VMEM((DMAS_PER_CHUNK, ROWS_PER_DMA), jnp.int32),
            pltpu.VMEM((DMAS_PER_CHUNK, ROWS_PER_DMA), jnp.int32),
            pltpu.VMEM((DMAS_PER_CHUNK, ROWS_PER_DMA, DIM), jnp.float32),
            pltpu.SemaphoreType.DMA,
        ],
        compiler_params=pltpu.CompilerParams(use_tc_tiling_on_sc=False),
    )
    def k(ids_hbm, table_hbm, off_hbm, out_hbm, idx_v, off_v, rows_v, sem):
        wid = lax.axis_index("s") * NUM_CORES + lax.axis_index("c")
        pltpu.sync_copy(off_hbm, off_v)

        def body(c, carry):
            pltpu.sync_copy(ids_hbm.at[wid, c], idx_v)
            for i in range(DMAS_PER_CHUNK):
                for j in range(ROWS_PER_DMA // LANES):
                    sl = pl.ds(j * LANES, LANES)
                    idx_v[i, sl] = idx_v[i, sl] + off_v[i, sl]
            copies = [
                pltpu.async_copy(table_hbm.at[idx_v.at[i]], rows_v.at[i], sem)
                for i in range(DMAS_PER_CHUNK)
            ]
            for cp in copies:
                cp.wait()
            pltpu.sync_copy(rows_v, out_hbm.at[wid, c])
            return carry

        lax.fori_loop(0, NCHUNKS, body, 0)

    return k(ids, table, off_tiled)


def kernel(hash_ids, table, offsets_buf):
    ids = hash_ids.reshape(NW, NCHUNKS, DMAS_PER_CHUNK, ROWS_PER_DMA)
    off_tiled = jnp.tile(offsets_buf, CHUNK // N_FIELDS).reshape(
        DMAS_PER_CHUNK, ROWS_PER_DMA
    )
    out = _sc_gather(ids, table, off_tiled)
    return out.reshape(BATCH, N_FIELDS, DIM)


# speedup vs baseline: 1.1783x; 1.1783x over previous
import functools

import jax
import jax.numpy as jnp
from jax import lax
from jax.experimental import pallas as pl
from jax.experimental.pallas import tpu as pltpu
from jax.experimental.pallas import tpu_sc as plsc

BATCH = 16384
N_FIELDS = 26
DIM = 16
LANES = 16

NUM_CORES = 2
NUM_SUBCORES = 16
NW = NUM_CORES * NUM_SUBCORES
BT = 128
NTILES = BATCH // BT
TILES_PER_W = NTILES // NW


def _sc_gather(ids, table, offsets_buf):
    mesh = plsc.VectorSubcoreMesh(core_axis_name="c", subcore_axis_name="s")

    @functools.partial(
        pl.kernel,
        mesh=mesh,
        out_type=jax.ShapeDtypeStruct(
            (N_FIELDS, DIM // 8, BT, 8, BT), jnp.float32
        ),
        scratch_types=[
            pltpu.VMEM((BT, N_FIELDS), jnp.int32),
            pltpu.VMEM((N_FIELDS, BT), jnp.int32),
            pltpu.VMEM((N_FIELDS, BT, DIM), jnp.float32),
            pltpu.VMEM((N_FIELDS, DIM // 8, 8, BT), jnp.float32),
            pltpu.VMEM((N_FIELDS, LANES), jnp.int32),
            pltpu.SemaphoreType.DMA,
        ],
        compiler_params=pltpu.CompilerParams(
            use_tc_tiling_on_sc=False, needs_layout_passes=False
        ),
    )
    def k(ids_hbm, table_hbm, off_hbm, out_hbm, idx2d, idxf, rows_v, obuf,
          off_v, sem):
        wid = lax.axis_index("s") * NUM_CORES + lax.axis_index("c")
        pltpu.sync_copy(off_hbm, off_v)
        iota = jax.lax.iota(jnp.int32, LANES)
        dt_idx = jax.lax.shift_right_logical(iota, 3)
        ds_idx = jax.lax.bitwise_and(iota, 7)

        def tile_body(t, carry):
            bt = wid * TILES_PER_W + t
            pltpu.sync_copy(ids_hbm.at[pl.ds(bt * BT, BT), :], idx2d)
            for f in range(N_FIELDS):
                fsplat = jnp.full((LANES,), f, dtype=jnp.int32)
                offf = off_v[f, :]
                for kk in range(BT // LANES):
                    bidx = iota + (kk * LANES)
                    g = plsc.load_gather(idx2d, [bidx, fsplat])
                    idxf[f, pl.ds(kk * LANES, LANES)] = g + offf
            copies = [
                pltpu.async_copy(table_hbm.at[idxf.at[f]], rows_v.at[f], sem)
                for f in range(N_FIELDS)
            ]
            for cp in copies:
                cp.wait()

            def b_body(b, c2):
                bs = jnp.full((LANES,), b, dtype=jnp.int32)
                for f in range(N_FIELDS):
                    fs = jnp.full((LANES,), f, dtype=jnp.int32)
                    row = rows_v[f, b, :]
                    plsc.store_scatter(obuf, [fs, dt_idx, ds_idx, bs], row)
                return c2

            lax.fori_loop(0, BT, b_body, 0)
            pltpu.sync_copy(obuf, out_hbm.at[:, :, bt, :, :])
            return carry

        lax.fori_loop(0, TILES_PER_W, tile_body, 0)

    return k(ids, table, offsets_buf)


def kernel(hash_ids, table, offsets_buf):
    off_b = jnp.broadcast_to(offsets_buf[:, None], (N_FIELDS, LANES))
    out5 = _sc_gather(hash_ids, table, off_b)
    return jnp.transpose(out5, (2, 4, 0, 1, 3)).reshape(BATCH, N_FIELDS, DIM)
